# baseline (speedup 1.0000x reference)
import jax
import jax.numpy as jnp
from jax import lax
from jax.experimental import pallas as pl
from jax.experimental.pallas import tpu as pltpu

_B, _T, _N, _H = 8, 512, 200, 128

_DN_TT = (((0,), (0,)), ((), ()))
_DN_NT = (((1,), (1,)), ((), ()))
_DN_NN = (((1,), (0,)), ((), ()))


def _mm(a, b, dn):
    return lax.dot_general(a, b, dn, preferred_element_type=jnp.float32)


def _body(X_ref, eps1_ref, W1a_ref, b1a_ref, g1a_ref, be1a_ref,
          W1b_ref, b1b_ref, g1b_ref, be1b_ref,
          eps2_ref, W2a_ref, b2a_ref, g2a_ref, be2a_ref,
          W2b_ref, b2b_ref, g2b_ref, be2b_ref, out_ref,
          mask_s, h_s):
    b = pl.program_id(0)

    x = X_ref[0]
    xm = x - jnp.mean(x, axis=0, keepdims=True)
    c = _mm(xm, xm, _DN_TT) / (_T - 1)
    ii = lax.broadcasted_iota(jnp.int32, (_N, _N), 0)
    jj = lax.broadcasted_iota(jnp.int32, (_N, _N), 1)
    ce = jnp.where(ii == jj, c, 0.0)
    d_row = jnp.sum(ce, axis=0, keepdims=True)
    d_col = jnp.sum(ce, axis=1, keepdims=True)
    c = c * lax.rsqrt(d_col) * lax.rsqrt(d_row)
    c = jnp.clip(c, -1.0, 1.0)
    c = jnp.where(jnp.isnan(c), 0.0, c)
    v = jnp.abs(c)
    mask = (c != 0.0).astype(jnp.float32)
    mask_s[b] = mask
    agg = _mm(mask, v, _DN_NN) + eps1_ref[0, 0] * v
    h_s[b] = _mm(agg, W1a_ref[...], _DN_NT) + b1a_ref[...]

    @pl.when(b == _B - 1)
    def _tail():
        inv_rows = 1.0 / (_B * _N)
        eps2 = eps2_ref[0, 0]

        def bn_relu(hs, g, be):
            m = sum(jnp.sum(h, axis=0, keepdims=True) for h in hs) * inv_rows
            var = sum(jnp.sum((h - m) * (h - m), axis=0, keepdims=True)
                      for h in hs) * inv_rows
            scale = g * lax.rsqrt(var + 1e-5)
            return [jnp.maximum((h - m) * scale + be, 0.0) for h in hs]

        h1 = bn_relu([h_s[i] for i in range(_B)], g1a_ref[...], be1a_ref[...])
        h1 = [_mm(h, W1b_ref[...], _DN_NT) + b1b_ref[...] for h in h1]
        x1 = bn_relu(h1, g1b_ref[...], be1b_ref[...])

        h2 = [_mm(mask_s[i], x1[i], _DN_NN) + eps2 * x1[i] for i in range(_B)]
        h2 = [_mm(h, W2a_ref[...], _DN_NT) + b2a_ref[...] for h in h2]
        h2 = bn_relu(h2, g2a_ref[...], be2a_ref[...])
        h2 = [_mm(h, W2b_ref[...], _DN_NT) + b2b_ref[...] for h in h2]
        x2 = bn_relu(h2, g2b_ref[...], be2b_ref[...])
        for i in range(_B):
            out_ref[i, :, :] = x2[i]


def kernel(X, eps1, W1a, b1a, g1a, be1a, W1b, b1b, g1b, be1b,
           eps2, W2a, b2a, g2a, be2a, W2b, b2b, g2b, be2b):
    r = lambda v: jnp.reshape(v, (1, -1))
    full = lambda s: pl.BlockSpec(s, lambda b: (0,) * len(s))
    return pl.pallas_call(
        _body,
        grid=(_B,),
        in_specs=[
            pl.BlockSpec((1, _T, _N), lambda b: (b, 0, 0)),
            full((1, 1)),
            full((_H, _N)), full((1, _H)), full((1, _H)), full((1, _H)),
            full((_H, _H)), full((1, _H)), full((1, _H)), full((1, _H)),
            full((1, 1)),
            full((_H, _H)), full((1, _H)), full((1, _H)), full((1, _H)),
            full((_H, _H)), full((1, _H)), full((1, _H)), full((1, _H)),
        ],
        out_specs=full((_B, _N, _H)),
        out_shape=jax.ShapeDtypeStruct((_B, _N, _H), jnp.float32),
        scratch_shapes=[
            pltpu.VMEM((_B, _N, _N), jnp.float32),
            pltpu.VMEM((_B, _N, _H), jnp.float32),
        ],
        compiler_params=pltpu.CompilerParams(
            dimension_semantics=("arbitrary",),
            vmem_limit_bytes=100 * 1024 * 1024),
    )(X, eps1, W1a, r(b1a), r(g1a), r(be1a), W1b, r(b1b), r(g1b), r(be1b),
      eps2, W2a, r(b2a), r(g2a), r(be2a), W2b, r(b2b), r(g2b), r(be2b))


# speedup vs baseline: 1.2521x; 1.2521x over previous
import jax
import jax.numpy as jnp
from jax import lax
from jax.experimental import pallas as pl
from jax.experimental.pallas import tpu as pltpu

_B, _T, _N, _H = 8, 512, 200, 128

_DN_TT = (((0,), (0,)), ((), ()))
_DN_NT = (((1,), (1,)), ((), ()))
_DN_NN = (((1,), (0,)), ((), ()))


def _mm(a, b, dn):
    return lax.dot_general(a, b, dn, preferred_element_type=jnp.float32)


def _body(X_ref, eps1_ref, W1a_ref, b1a_ref, g1a_ref, be1a_ref,
          W1b_ref, b1b_ref, g1b_ref, be1b_ref,
          eps2_ref, W2a_ref, b2a_ref, g2a_ref, be2a_ref,
          W2b_ref, b2b_ref, g2b_ref, be2b_ref, out_ref):
    eps1 = eps1_ref[0, 0]
    eps2 = eps2_ref[0, 0]

    ii = lax.broadcasted_iota(jnp.int32, (_N, _N), 0)
    jj = lax.broadcasted_iota(jnp.int32, (_N, _N), 1)
    on_diag = ii == jj
    vs, masks = [], []
    for b in range(_B):
        x = X_ref[b]
        xm = x - jnp.mean(x, axis=0, keepdims=True)
        c = _mm(xm, xm, _DN_TT) / (_T - 1)
        ce = jnp.where(on_diag, c, 0.0)
        d_row = jnp.sum(ce, axis=0, keepdims=True)
        d_col = jnp.sum(ce, axis=1, keepdims=True)
        c = c * lax.rsqrt(d_col) * lax.rsqrt(d_row)
        c = jnp.clip(c, -1.0, 1.0)
        c = jnp.where(jnp.isnan(c), 0.0, c)
        vs.append(jnp.abs(c))
        masks.append((c != 0.0).astype(jnp.float32))

    def gin(feats, eps, Wa, ba, ga, bea, Wb, bb, gb, beb):
        h1 = [_mm(_mm(masks[b], feats[b], _DN_NN) + eps * feats[b],
                  Wa, _DN_NT) + ba for b in range(_B)]
        inv_rows = 1.0 / (_B * _N)

        def bn_relu(hs, g, be):
            m = sum(jnp.sum(h, axis=0, keepdims=True) for h in hs) * inv_rows
            var = sum(jnp.sum((h - m) * (h - m), axis=0, keepdims=True)
                      for h in hs) * inv_rows
            scale = g * lax.rsqrt(var + 1e-5)
            return [jnp.maximum((h - m) * scale + be, 0.0) for h in hs]

        h1 = bn_relu(h1, ga, bea)
        h2 = [_mm(h, Wb, _DN_NT) + bb for h in h1]
        return bn_relu(h2, gb, beb)

    x1 = gin(vs, eps1, W1a_ref[...], b1a_ref[...], g1a_ref[...], be1a_ref[...],
             W1b_ref[...], b1b_ref[...], g1b_ref[...], be1b_ref[...])
    x2 = gin(x1, eps2, W2a_ref[...], b2a_ref[...], g2a_ref[...], be2a_ref[...],
             W2b_ref[...], b2b_ref[...], g2b_ref[...], be2b_ref[...])
    for b in range(_B):
        out_ref[b, :, :] = x2[b]


def kernel(X, eps1, W1a, b1a, g1a, be1a, W1b, b1b, g1b, be1b,
           eps2, W2a, b2a, g2a, be2a, W2b, b2b, g2b, be2b):
    r = lambda v: jnp.reshape(v, (1, -1))
    return pl.pallas_call(
        _body,
        out_shape=jax.ShapeDtypeStruct((_B, _N, _H), jnp.float32),
        compiler_params=pltpu.CompilerParams(
            vmem_limit_bytes=100 * 1024 * 1024),
    )(X, eps1, W1a, r(b1a), r(g1a), r(be1a), W1b, r(b1b), r(g1b), r(be1b),
      eps2, W2a, r(b2a), r(g2a), r(be2a), W2b, r(b2b), r(g2b), r(be2b))
